# baseline (speedup 1.0000x reference)
import functools

import jax
import jax.numpy as jnp
from jax import lax
from jax.experimental import pallas as pl
from jax.experimental.pallas import tpu as pltpu
from jax.experimental.pallas import tpu_sc as plsc


_CHUNK = 32


@functools.lru_cache(maxsize=None)
def _make_sc_copy(T, D):
    info = plsc.get_sparse_core_info()
    nw = info.num_cores * info.num_subcores
    rows_per_w = T // nw
    chunk = min(_CHUNK, rows_per_w)
    nchunks = rows_per_w // chunk
    npairs = nchunks // 2
    assert T % nw == 0 and rows_per_w % chunk == 0 and nchunks % 2 == 0

    @functools.partial(
        pl.kernel,
        mesh=plsc.VectorSubcoreMesh(core_axis_name="c", subcore_axis_name="s"),
        out_type=jax.ShapeDtypeStruct((1, T, D), jnp.float32),
        scratch_types=[
            pltpu.VMEM((2, chunk, D), jnp.float32),
            pltpu.SemaphoreType.DMA((2,)),
            pltpu.SemaphoreType.DMA((2,)),
        ],
    )
    def sc_copy(emb_hbm, out_hbm, buf, in_sems, out_sems):
        wid = lax.axis_index("s") * info.num_cores + lax.axis_index("c")
        base = wid * rows_per_w

        def in_copy(i, slot):
            return pltpu.make_async_copy(
                emb_hbm.at[pl.ds(base + i * chunk, chunk), :],
                buf.at[slot],
                in_sems.at[slot],
            )

        def out_copy(i, slot):
            return pltpu.make_async_copy(
                buf.at[slot],
                out_hbm.at[0, pl.ds(base + i * chunk, chunk), :],
                out_sems.at[slot],
            )

        in_copy(0, 0).start()

        def body(k, carry):
            a = 2 * k
            b = a + 1

            @pl.when(k > 0)
            def _():
                out_copy(a - 1, 1).wait()

            in_copy(b, 1).start()
            in_copy(a, 0).wait()
            out_copy(a, 0).start()
            in_copy(b, 1).wait()
            out_copy(b, 1).start()

            @pl.when(k + 1 < npairs)
            def _():
                out_copy(a, 0).wait()
                in_copy(a + 2, 0).start()

            return carry

        lax.fori_loop(0, npairs, body, 0)
        out_copy(2 * npairs - 2, 0).wait()
        out_copy(2 * npairs - 1, 1).wait()

    return sc_copy


def kernel(x, emb):
    del x
    T, D = emb.shape
    return _make_sc_copy(T, D)(emb)


# speedup vs baseline: 1.0477x; 1.0477x over previous
import functools

import jax
import jax.numpy as jnp
from jax import lax
from jax.experimental import pallas as pl
from jax.experimental.pallas import tpu as pltpu
from jax.experimental.pallas import tpu_sc as plsc


_CHUNK = 32
_NBUF = 2


@functools.lru_cache(maxsize=None)
def _make_sc_copy(T, D):
    info = plsc.get_sparse_core_info()
    nw = info.num_cores * info.num_subcores
    rows_per_w = T // nw
    chunk = min(_CHUNK, rows_per_w)
    nchunks = rows_per_w // chunk
    assert T % nw == 0 and rows_per_w % chunk == 0

    @functools.partial(
        pl.kernel,
        mesh=plsc.VectorSubcoreMesh(core_axis_name="c", subcore_axis_name="s"),
        out_type=jax.ShapeDtypeStruct((1, T, D), jnp.float32),
        scratch_types=[
            pltpu.VMEM((_NBUF, chunk, D), jnp.float32),
            pltpu.SemaphoreType.DMA((_NBUF,)),
            pltpu.SemaphoreType.DMA((_NBUF,)),
        ],
    )
    def sc_copy(emb_hbm, out_hbm, buf, in_sems, out_sems):
        wid = lax.axis_index("s") * info.num_cores + lax.axis_index("c")
        base = wid * rows_per_w

        def in_copy(i, slot):
            return pltpu.make_async_copy(
                emb_hbm.at[pl.ds(base + i * chunk, chunk), :],
                buf.at[slot],
                in_sems.at[slot],
            )

        def out_copy(i, slot):
            return pltpu.make_async_copy(
                buf.at[slot],
                out_hbm.at[0, pl.ds(base + i * chunk, chunk), :],
                out_sems.at[slot],
            )

        in_copy(0, 0).start()
        for i in range(nchunks):
            slot = i % _NBUF
            if i + 1 < nchunks:
                nslot = (i + 1) % _NBUF
                if i + 1 >= _NBUF:
                    out_copy(i + 1 - _NBUF, nslot).wait()
                in_copy(i + 1, nslot).start()
            in_copy(i, slot).wait()
            out_copy(i, slot).start()
        for i in range(max(0, nchunks - _NBUF), nchunks):
            out_copy(i, i % _NBUF).wait()

    return sc_copy


def kernel(x, emb):
    del x
    T, D = emb.shape
    return _make_sc_copy(T, D)(emb)


# speedup vs baseline: 1.0657x; 1.0172x over previous
import functools

import jax
import jax.numpy as jnp
from jax import lax
from jax.experimental import pallas as pl
from jax.experimental.pallas import tpu as pltpu
from jax.experimental.pallas import tpu_sc as plsc


_CHUNK = 32


@functools.lru_cache(maxsize=None)
def _make_sc_copy(T, D):
    info = plsc.get_sparse_core_info()
    nc, ns = info.num_cores, info.num_subcores
    nw = nc * ns
    rows_per_w = T // nw
    chunk = min(_CHUNK, rows_per_w)
    nchunks = rows_per_w // chunk
    npairs = nchunks // 2
    assert T % nw == 0 and rows_per_w % chunk == 0 and nchunks % 2 == 0

    @functools.partial(
        pl.kernel,
        mesh=plsc.VectorSubcoreMesh(core_axis_name="c", subcore_axis_name="s"),
        out_type=jax.ShapeDtypeStruct((1, T, D), jnp.float32),
        scratch_types=[
            pltpu.VMEM((2, chunk, D), jnp.float32),
            pltpu.VMEM_SHARED((ns, 2, chunk, D), jnp.float32),
            pltpu.SemaphoreType.DMA((2,)),
            pltpu.SemaphoreType.DMA((2,)),
            pltpu.SemaphoreType.DMA((2,)),
            pltpu.SemaphoreType.DMA((2,)),
        ],
    )
    def sc_copy(emb_hbm, out_hbm, tbuf, sbuf, tin, tout, sin, sout):
        cid = lax.axis_index("c")
        sid = lax.axis_index("s")
        wid = sid * nc + cid
        base = wid * rows_per_w

        def t_in(i, slot):
            return pltpu.make_async_copy(
                emb_hbm.at[pl.ds(base + i * chunk, chunk), :],
                tbuf.at[slot], tin.at[slot])

        def t_out(i, slot):
            return pltpu.make_async_copy(
                tbuf.at[slot],
                out_hbm.at[0, pl.ds(base + i * chunk, chunk), :],
                tout.at[slot])

        def s_in(i, slot):
            return pltpu.make_async_copy(
                emb_hbm.at[pl.ds(base + i * chunk, chunk), :],
                sbuf.at[sid, slot], sin.at[slot])

        def s_out(i, slot):
            return pltpu.make_async_copy(
                sbuf.at[sid, slot],
                out_hbm.at[0, pl.ds(base + i * chunk, chunk), :],
                sout.at[slot])

        t_in(0, 0).start()
        s_in(1, 0).start()
        for k in range(npairs):
            a, b = 2 * k, 2 * k + 1
            if k + 1 < npairs:
                if k >= 1:
                    t_out(a - 2, (k + 1) % 2).wait()
                    s_out(b - 2, (k + 1) % 2).wait()
                t_in(a + 2, (k + 1) % 2).start()
                s_in(b + 2, (k + 1) % 2).start()
            t_in(a, k % 2).wait()
            t_out(a, k % 2).start()
            s_in(b, k % 2).wait()
            s_out(b, k % 2).start()
        for k in range(max(0, npairs - 2), npairs):
            t_out(2 * k, k % 2).wait()
            s_out(2 * k + 1, k % 2).wait()

    return sc_copy


def kernel(x, emb):
    del x
    T, D = emb.shape
    return _make_sc_copy(T, D)(emb)
